# baseline (speedup 1.0000x reference)
import functools

import jax
import jax.numpy as jnp
from jax import lax
from jax.experimental import pallas as pl
from jax.experimental.pallas import tpu as pltpu
from jax.experimental.pallas import tpu_sc as plsc

N = 2097152
NC = 2
NS = 16
NW = NC * NS
CHUNK = N // NW
LANES = 16
HALF = CHUNK // 2
SUB = CHUNK // 4
SS = 128
NG = 8
NREG = CHUNK // SS
REG_H = HALF // SS

_mesh = plsc.VectorSubcoreMesh(core_axis_name="c", subcore_axis_name="s")
_params = pltpu.CompilerParams(needs_layout_passes=False)


def _wid():
    return lax.axis_index("c") * NS + lax.axis_index("s")


@functools.partial(
    pl.kernel,
    out_type=(
        jax.ShapeDtypeStruct((NW * LANES,), jnp.float32),
        jax.ShapeDtypeStruct((NW * NREG,), jnp.float32),
    ),
    mesh=_mesh,
    compiler_params=_params,
    scratch_types=[
        pltpu.VMEM((HALF,), jnp.float32),
        pltpu.VMEM((HALF,), jnp.float32),
        pltpu.VMEM((LANES,), jnp.float32),
        pltpu.VMEM((NREG,), jnp.float32),
        pltpu.SemaphoreType.DMA,
        pltpu.SemaphoreType.DMA,
    ],
)
def _chunk_sums(x_hbm, out_hbm, tot_hbm, buf0, buf1, accv, totv, sem0, sem1):
    wid = _wid()
    base = wid * CHUNK
    bufs = (buf0, buf1)
    sems = (sem0, sem1)
    copies = [
        pltpu.async_copy(x_hbm.at[pl.ds(base + h * HALF, HALF)], bufs[h], sems[h])
        for h in range(2)
    ]

    lane = lax.iota(jnp.int32, LANES)
    mask0 = lane == 0
    acc_glob = jnp.zeros((LANES,), jnp.float32)
    for h in range(2):
        copies[h].wait()
        buf = bufs[h]

        @plsc.parallel_loop(0, REG_H, carry=acc_glob)
        def rbody(j, ag):
            o = j * SS
            a0 = buf[pl.ds(o + 0 * LANES, LANES)]
            a1 = buf[pl.ds(o + 1 * LANES, LANES)]
            a2 = buf[pl.ds(o + 2 * LANES, LANES)]
            a3 = buf[pl.ds(o + 3 * LANES, LANES)]
            a0 = a0 + buf[pl.ds(o + 4 * LANES, LANES)]
            a1 = a1 + buf[pl.ds(o + 5 * LANES, LANES)]
            a2 = a2 + buf[pl.ds(o + 6 * LANES, LANES)]
            a3 = a3 + buf[pl.ds(o + 7 * LANES, LANES)]
            ra = (a0 + a1) + (a2 + a3)
            t = jnp.sum(ra)
            plsc.store_scatter(
                totv,
                [jnp.full((LANES,), h * REG_H + j, jnp.int32)],
                jnp.full((LANES,), t),
                mask=mask0,
            )
            return ag + ra

        acc_glob = rbody

    accv[...] = acc_glob
    pltpu.sync_copy(accv, out_hbm.at[pl.ds(wid * LANES, LANES)])
    pltpu.sync_copy(totv, tot_hbm.at[pl.ds(wid * NREG, NREG)])


@functools.partial(
    pl.kernel,
    out_type=jax.ShapeDtypeStruct((N,), jnp.float32),
    mesh=_mesh,
    compiler_params=_params,
    scratch_types=[
        pltpu.VMEM((SUB,), jnp.float32),
        pltpu.VMEM((SUB,), jnp.float32),
        pltpu.VMEM((SUB,), jnp.float32),
        pltpu.VMEM((SUB,), jnp.float32),
        pltpu.VMEM((NW * LANES,), jnp.float32),
        pltpu.VMEM((NREG,), jnp.float32),
        pltpu.SemaphoreType.DMA,
        pltpu.SemaphoreType.DMA,
        pltpu.SemaphoreType.DMA,
        pltpu.SemaphoreType.DMA,
    ],
)
def _scan_chunks(x_hbm, sums_hbm, tots_hbm, out_hbm, in0, in1, out0, out1,
                 sums_v, tot_v, isem0, isem1, osem0, osem1):
    wid = _wid()
    base = wid * CHUNK
    ins = (in0, in1)
    outs = (out0, out1)
    isems = (isem0, isem1)
    osems = (osem0, osem1)

    in_copies = [None] * 4
    out_copies = [None] * 4
    for b in range(2):
        in_copies[b] = pltpu.async_copy(
            x_hbm.at[pl.ds(base + b * SUB, SUB)], ins[b], isems[b])

    pltpu.sync_copy(sums_hbm, sums_v)
    pltpu.sync_copy(tots_hbm.at[pl.ds(wid * NREG, NREG)], tot_v)

    def off_body(w, acc):
        v = sums_v[pl.ds(w * LANES, LANES)]
        keep = (w < wid).astype(jnp.float32)
        return acc + v * keep

    offv = lax.fori_loop(0, NW, off_body, jnp.zeros((LANES,), jnp.float32))
    carry = jnp.sum(offv) - 1.0

    lane = lax.iota(jnp.int32, LANES)

    for b in range(4):
        in_copies[b].wait()
        if b >= 2:
            out_copies[b - 2].wait()
        ibuf = ins[b % 2]
        obuf = outs[b % 2]

        rt = [tot_v[pl.ds(b * NG * LANES + q * LANES, LANES)] for q in range(NG)]
        rs = [plsc.cumsum(r) for r in rt]
        bases = [carry]
        for q in range(1, NG):
            bases.append(bases[q - 1] + rs[q - 1][15])
        runs = tuple((rs[q] - rt[q]) + bases[q] for q in range(NG))
        carry = bases[NG - 1] + rs[NG - 1][15]
        ibase = [(lane + q * LANES) * SS - lane for q in range(NG)]

        def edge(k, rc, m):
            new = []
            for q in range(NG):
                idx = ibase[q] + k
                v = plsc.load_gather(ibuf, [idx], mask=m)
                r2 = rc[q] + jnp.where(m, v, 0.0)
                plsc.store_scatter(obuf, [idx], r2, mask=m)
                new.append(r2)
            return tuple(new)

        runs = lax.fori_loop(
            0, LANES - 1, lambda k, rc: edge(k, rc, lane <= k), runs)

        @plsc.parallel_loop(LANES - 1, SS, unroll=4, carry=runs)
        def mbody(k, rc):
            new = []
            for q in range(NG):
                idx = ibase[q] + k
                v = plsc.load_gather(ibuf, [idx])
                r2 = rc[q] + v
                plsc.store_scatter(obuf, [idx], r2)
                new.append(r2)
            return tuple(new)

        runs = lax.fori_loop(
            SS, SS + LANES - 1, lambda k, rc: edge(k, rc, lane > k - SS), mbody)

        out_copies[b] = pltpu.async_copy(
            obuf, out_hbm.at[pl.ds(base + b * SUB, SUB)], osems[b % 2])
        if b + 2 < 4:
            in_copies[b + 2] = pltpu.async_copy(
                x_hbm.at[pl.ds(base + (b + 2) * SUB, SUB)], ins[b % 2], isems[b % 2])

    out_copies[2].wait()
    out_copies[3].wait()


def kernel(mask_i):
    sums, tots = _chunk_sums(mask_i)
    return _scan_chunks(mask_i, sums, tots)


# speedup vs baseline: 1.0892x; 1.0892x over previous
import functools

import jax
import jax.numpy as jnp
from jax import lax
from jax.experimental import pallas as pl
from jax.experimental.pallas import tpu as pltpu
from jax.experimental.pallas import tpu_sc as plsc

N = 2097152
NC = 2
NS = 16
NW = NC * NS
CHUNK = N // NW
LANES = 16
HALF = CHUNK // 2
SUB = CHUNK // 4
SS = 256
NG = 4
NREG = CHUNK // SS
REG_H = HALF // SS

_mesh = plsc.VectorSubcoreMesh(core_axis_name="c", subcore_axis_name="s")
_params = pltpu.CompilerParams(needs_layout_passes=False)


def _wid():
    return lax.axis_index("c") * NS + lax.axis_index("s")


@functools.partial(
    pl.kernel,
    out_type=(
        jax.ShapeDtypeStruct((NW * LANES,), jnp.float32),
        jax.ShapeDtypeStruct((NW * NREG,), jnp.float32),
    ),
    mesh=_mesh,
    compiler_params=_params,
    scratch_types=[
        pltpu.VMEM((HALF,), jnp.float32),
        pltpu.VMEM((HALF,), jnp.float32),
        pltpu.VMEM((LANES,), jnp.float32),
        pltpu.VMEM((NREG,), jnp.float32),
        pltpu.SemaphoreType.DMA,
        pltpu.SemaphoreType.DMA,
    ],
)
def _chunk_sums(x_hbm, out_hbm, tot_hbm, buf0, buf1, accv, totv, sem0, sem1):
    wid = _wid()
    base = wid * CHUNK
    bufs = (buf0, buf1)
    sems = (sem0, sem1)
    copies = [
        pltpu.async_copy(x_hbm.at[pl.ds(base + h * HALF, HALF)], bufs[h], sems[h])
        for h in range(2)
    ]

    lane = lax.iota(jnp.int32, LANES)
    mask0 = lane == 0
    acc_glob = jnp.zeros((LANES,), jnp.float32)
    for h in range(2):
        copies[h].wait()
        buf = bufs[h]

        @plsc.parallel_loop(0, REG_H, carry=acc_glob)
        def rbody(j, ag):
            o = j * SS
            a0 = buf[pl.ds(o + 0 * LANES, LANES)]
            a1 = buf[pl.ds(o + 1 * LANES, LANES)]
            a2 = buf[pl.ds(o + 2 * LANES, LANES)]
            a3 = buf[pl.ds(o + 3 * LANES, LANES)]
            for t in range(4, SS // LANES):
                a0, a1, a2, a3 = (
                    a0 + buf[pl.ds(o + t * LANES, LANES)] if t % 4 == 0 else a0,
                    a1 + buf[pl.ds(o + t * LANES, LANES)] if t % 4 == 1 else a1,
                    a2 + buf[pl.ds(o + t * LANES, LANES)] if t % 4 == 2 else a2,
                    a3 + buf[pl.ds(o + t * LANES, LANES)] if t % 4 == 3 else a3,
                )
            ra = (a0 + a1) + (a2 + a3)
            t = jnp.sum(ra)
            plsc.store_scatter(
                totv,
                [jnp.full((LANES,), h * REG_H + j, jnp.int32)],
                jnp.full((LANES,), t),
                mask=mask0,
            )
            return ag + ra

        acc_glob = rbody

    accv[...] = acc_glob
    pltpu.sync_copy(accv, out_hbm.at[pl.ds(wid * LANES, LANES)])
    pltpu.sync_copy(totv, tot_hbm.at[pl.ds(wid * NREG, NREG)])


@functools.partial(
    pl.kernel,
    out_type=jax.ShapeDtypeStruct((N,), jnp.float32),
    mesh=_mesh,
    compiler_params=_params,
    scratch_types=[
        pltpu.VMEM((SUB,), jnp.float32),
        pltpu.VMEM((SUB,), jnp.float32),
        pltpu.VMEM((SUB,), jnp.float32),
        pltpu.VMEM((SUB,), jnp.float32),
        pltpu.VMEM((NW * LANES,), jnp.float32),
        pltpu.VMEM((NREG,), jnp.float32),
        pltpu.SemaphoreType.DMA,
        pltpu.SemaphoreType.DMA,
        pltpu.SemaphoreType.DMA,
        pltpu.SemaphoreType.DMA,
    ],
)
def _scan_chunks(x_hbm, sums_hbm, tots_hbm, out_hbm, in0, in1, out0, out1,
                 sums_v, tot_v, isem0, isem1, osem0, osem1):
    wid = _wid()
    base = wid * CHUNK
    ins = (in0, in1)
    outs = (out0, out1)
    isems = (isem0, isem1)
    osems = (osem0, osem1)

    in_copies = [None] * 4
    out_copies = [None] * 4
    for b in range(2):
        in_copies[b] = pltpu.async_copy(
            x_hbm.at[pl.ds(base + b * SUB, SUB)], ins[b], isems[b])

    pltpu.sync_copy(sums_hbm, sums_v)
    pltpu.sync_copy(tots_hbm.at[pl.ds(wid * NREG, NREG)], tot_v)

    def off_body(w, acc):
        v = sums_v[pl.ds(w * LANES, LANES)]
        keep = (w < wid).astype(jnp.float32)
        return acc + v * keep

    offv = lax.fori_loop(0, NW, off_body, jnp.zeros((LANES,), jnp.float32))
    carry = jnp.sum(offv) - 1.0

    lane = lax.iota(jnp.int32, LANES)

    for b in range(4):
        in_copies[b].wait()
        if b >= 2:
            out_copies[b - 2].wait()
        ibuf = ins[b % 2]
        obuf = outs[b % 2]

        rt = [tot_v[pl.ds(b * NG * LANES + q * LANES, LANES)] for q in range(NG)]
        rs = [plsc.cumsum(r) for r in rt]
        bases = [carry]
        for q in range(1, NG):
            bases.append(bases[q - 1] + rs[q - 1][15])
        runs = tuple((rs[q] - rt[q]) + bases[q] for q in range(NG))
        carry = bases[NG - 1] + rs[NG - 1][15]
        ibase = [(lane + q * LANES) * SS - lane for q in range(NG)]

        def edge(k, rc, m):
            new = []
            for q in range(NG):
                idx = ibase[q] + k
                v = plsc.load_gather(ibuf, [idx], mask=m)
                r2 = rc[q] + jnp.where(m, v, 0.0)
                plsc.store_scatter(obuf, [idx], r2, mask=m)
                new.append(r2)
            return tuple(new)

        runs = lax.fori_loop(
            0, LANES - 1, lambda k, rc: edge(k, rc, lane <= k), runs)

        @plsc.parallel_loop(LANES - 1, SS - 1, step=2, carry=runs)
        def mbody(k, rc):
            new = []
            for q in range(NG):
                i0 = ibase[q] + k
                i1 = ibase[q] + (k + 1)
                v0 = plsc.load_gather(ibuf, [i0])
                v1 = plsc.load_gather(ibuf, [i1])
                s0 = rc[q] + v0
                r2 = rc[q] + (v0 + v1)
                plsc.store_scatter(obuf, [i0], s0)
                plsc.store_scatter(obuf, [i1], r2)
                new.append(r2)
            return tuple(new)

        runs = lax.fori_loop(
            SS - 1, SS + LANES - 1, lambda k, rc: edge(k, rc, lane > k - SS), mbody)

        out_copies[b] = pltpu.async_copy(
            obuf, out_hbm.at[pl.ds(base + b * SUB, SUB)], osems[b % 2])
        if b + 2 < 4:
            in_copies[b + 2] = pltpu.async_copy(
                x_hbm.at[pl.ds(base + (b + 2) * SUB, SUB)], ins[b % 2], isems[b % 2])

    out_copies[2].wait()
    out_copies[3].wait()


def kernel(mask_i):
    sums, tots = _chunk_sums(mask_i)
    return _scan_chunks(mask_i, sums, tots)


# speedup vs baseline: 1.0908x; 1.0015x over previous
import functools

import jax
import jax.numpy as jnp
from jax import lax
from jax.experimental import pallas as pl
from jax.experimental.pallas import tpu as pltpu
from jax.experimental.pallas import tpu_sc as plsc

N = 2097152
NC = 2
NS = 16
NW = NC * NS
CHUNK = N // NW
LANES = 16
HALF = CHUNK // 2
SUB = CHUNK // 4
SS = 256
NG = 4
NREG = CHUNK // SS
REG_H = HALF // SS

_mesh = plsc.VectorSubcoreMesh(core_axis_name="c", subcore_axis_name="s")
_params = pltpu.CompilerParams(needs_layout_passes=False)


def _wid():
    return lax.axis_index("c") * NS + lax.axis_index("s")


@functools.partial(
    pl.kernel,
    out_type=(
        jax.ShapeDtypeStruct((NW * LANES,), jnp.float32),
        jax.ShapeDtypeStruct((NW * NREG,), jnp.float32),
    ),
    mesh=_mesh,
    compiler_params=_params,
    scratch_types=[
        pltpu.VMEM((HALF,), jnp.float32),
        pltpu.VMEM((HALF,), jnp.float32),
        pltpu.VMEM((LANES,), jnp.float32),
        pltpu.VMEM((NREG,), jnp.float32),
        pltpu.SemaphoreType.DMA,
        pltpu.SemaphoreType.DMA,
    ],
)
def _chunk_sums(x_hbm, out_hbm, tot_hbm, buf0, buf1, accv, totv, sem0, sem1):
    wid = _wid()
    base = wid * CHUNK
    bufs = (buf0, buf1)
    sems = (sem0, sem1)
    copies = [
        pltpu.async_copy(x_hbm.at[pl.ds(base + h * HALF, HALF)], bufs[h], sems[h])
        for h in range(2)
    ]

    lane = lax.iota(jnp.int32, LANES)
    mask0 = lane == 0
    acc_glob = jnp.zeros((LANES,), jnp.float32)
    for h in range(2):
        copies[h].wait()
        buf = bufs[h]

        @plsc.parallel_loop(0, REG_H, carry=acc_glob)
        def rbody(j, ag):
            o = j * SS
            a0 = buf[pl.ds(o + 0 * LANES, LANES)]
            a1 = buf[pl.ds(o + 1 * LANES, LANES)]
            a2 = buf[pl.ds(o + 2 * LANES, LANES)]
            a3 = buf[pl.ds(o + 3 * LANES, LANES)]
            for t in range(4, SS // LANES):
                a0, a1, a2, a3 = (
                    a0 + buf[pl.ds(o + t * LANES, LANES)] if t % 4 == 0 else a0,
                    a1 + buf[pl.ds(o + t * LANES, LANES)] if t % 4 == 1 else a1,
                    a2 + buf[pl.ds(o + t * LANES, LANES)] if t % 4 == 2 else a2,
                    a3 + buf[pl.ds(o + t * LANES, LANES)] if t % 4 == 3 else a3,
                )
            ra = (a0 + a1) + (a2 + a3)
            t = jnp.sum(ra)
            plsc.store_scatter(
                totv,
                [jnp.full((LANES,), h * REG_H + j, jnp.int32)],
                jnp.full((LANES,), t),
                mask=mask0,
            )
            return ag + ra

        acc_glob = rbody

    accv[...] = acc_glob
    pltpu.sync_copy(accv, out_hbm.at[pl.ds(wid * LANES, LANES)])
    pltpu.sync_copy(totv, tot_hbm.at[pl.ds(wid * NREG, NREG)])


@functools.partial(
    pl.kernel,
    out_type=jax.ShapeDtypeStruct((N,), jnp.float32),
    mesh=_mesh,
    compiler_params=_params,
    scratch_types=[
        pltpu.VMEM((SUB,), jnp.float32),
        pltpu.VMEM((SUB,), jnp.float32),
        pltpu.VMEM((SUB,), jnp.float32),
        pltpu.VMEM((SUB,), jnp.float32),
        pltpu.VMEM((NW * LANES,), jnp.float32),
        pltpu.VMEM((NREG,), jnp.float32),
        pltpu.SemaphoreType.DMA,
        pltpu.SemaphoreType.DMA,
        pltpu.SemaphoreType.DMA,
        pltpu.SemaphoreType.DMA,
    ],
)
def _scan_chunks(x_hbm, sums_hbm, tots_hbm, out_hbm, in0, in1, out0, out1,
                 sums_v, tot_v, isem0, isem1, osem0, osem1):
    wid = _wid()
    base = wid * CHUNK
    ins = (in0, in1)
    outs = (out0, out1)
    isems = (isem0, isem1)
    osems = (osem0, osem1)

    in_copies = [None] * 4
    out_copies = [None] * 4
    for b in range(2):
        in_copies[b] = pltpu.async_copy(
            x_hbm.at[pl.ds(base + b * SUB, SUB)], ins[b], isems[b])

    pltpu.sync_copy(sums_hbm, sums_v)
    pltpu.sync_copy(tots_hbm.at[pl.ds(wid * NREG, NREG)], tot_v)

    def off_body(w, acc):
        v = sums_v[pl.ds(w * LANES, LANES)]
        keep = (w < wid).astype(jnp.float32)
        return acc + v * keep

    offv = lax.fori_loop(0, NW, off_body, jnp.zeros((LANES,), jnp.float32))
    carry = jnp.sum(offv) - 1.0

    lane = lax.iota(jnp.int32, LANES)

    for b in range(4):
        in_copies[b].wait()
        if b >= 2:
            out_copies[b - 2].wait()
        ibuf = ins[b % 2]
        obuf = outs[b % 2]

        rt = [tot_v[pl.ds(b * NG * LANES + q * LANES, LANES)] for q in range(NG)]
        rs = [plsc.cumsum(r) for r in rt]
        bases = [carry]
        for q in range(1, NG):
            bases.append(bases[q - 1] + rs[q - 1][15])
        runs = tuple((rs[q] - rt[q]) + bases[q] for q in range(NG))
        carry = bases[NG - 1] + rs[NG - 1][15]
        ibase = [(lane + q * LANES) * SS - lane for q in range(NG)]

        def edge(k, rc, m):
            new = []
            for q in range(NG):
                idx = ibase[q] + k
                v = plsc.load_gather(ibuf, [idx], mask=m)
                r2 = rc[q] + jnp.where(m, v, 0.0)
                plsc.store_scatter(obuf, [idx], r2, mask=m)
                new.append(r2)
            return tuple(new)

        runs = lax.fori_loop(
            0, LANES - 1, lambda k, rc: edge(k, rc, lane <= k), runs)

        @plsc.parallel_loop(LANES - 1, SS, carry=runs)
        def mbody(k, rc):
            new = []
            for q in range(NG):
                idx = ibase[q] + k
                v = plsc.load_gather(ibuf, [idx])
                r2 = rc[q] + v
                plsc.store_scatter(obuf, [idx], r2)
                new.append(r2)
            return tuple(new)

        runs = lax.fori_loop(
            SS, SS + LANES - 1, lambda k, rc: edge(k, rc, lane > k - SS), mbody)

        out_copies[b] = pltpu.async_copy(
            obuf, out_hbm.at[pl.ds(base + b * SUB, SUB)], osems[b % 2])
        if b + 2 < 4:
            in_copies[b + 2] = pltpu.async_copy(
                x_hbm.at[pl.ds(base + (b + 2) * SUB, SUB)], ins[b % 2], isems[b % 2])

    out_copies[2].wait()
    out_copies[3].wait()


def kernel(mask_i):
    sums, tots = _chunk_sums(mask_i)
    return _scan_chunks(mask_i, sums, tots)
